# Initial kernel scaffold; baseline (speedup 1.0000x reference)
import jax
import jax.numpy as jnp
from jax.experimental import pallas as pl


def kernel(x, noise, Wg, bg, Wn, bn, W1, W3, W2):
    raise NotImplementedError("write your pallas kernel here")


# speedup vs baseline: 2.1373x; 2.1373x over previous
import functools
import jax, jax.numpy as jnp
from jax import lax
from jax.experimental import pallas as pl
from jax.experimental.pallas import tpu as pltpu
from jax.experimental.pallas import tpu_sc as plsc

B, S, D = 2, 2048, 1024
E, TOPK = 8, 2
HID = 2816
T = B * S
CAP = T * TOPK // E
P = T * TOPK
TB = 512
NTB = T // TB
HB = 256
NHB = HID // HB

NSC_C, NSC_S, LANES = 2, 16, 16
NW = NSC_C * NSC_S
ROWS_W = P // NW
SCAN_CH = 512
N_SCAN = P // SCAN_CH
GCH = 64
TOK_W = T // NW
CCH = 32

_SC_MESH = plsc.VectorSubcoreMesh(core_axis_name="c", subcore_axis_name="s")


def _router_body(x_ref, noise_ref, wg_ref, bg_ref, wn_ref, bn_ref,
                 d1_ref, d2_ref, w1_ref, w2_ref, carry_ref):
    i = pl.program_id(0)

    @pl.when(i == 0)
    def _():
        carry_ref[...] = jnp.zeros_like(carry_ref)

    x = x_ref[...]
    lg = jnp.dot(x, wg_ref[...].T, preferred_element_type=jnp.float32) + bg_ref[...]
    ln = jnp.dot(x, wn_ref[...].T, preferred_element_type=jnp.float32) + bn_ref[...]
    sp = jnp.maximum(ln, 0.0) + jnp.log1p(jnp.exp(-jnp.abs(ln)))
    nz = lg + noise_ref[...] * sp

    ei = lax.broadcasted_iota(jnp.int32, (TB, E), 1)
    v1 = jnp.max(nz, axis=1, keepdims=True)
    e1 = jnp.min(jnp.where(nz == v1, ei, E), axis=1, keepdims=True)
    m1 = ei == e1
    nz2 = jnp.where(m1, -jnp.inf, nz)
    v2 = jnp.max(nz2, axis=1, keepdims=True)
    e2 = jnp.min(jnp.where(nz2 == v2, ei, E), axis=1, keepdims=True)
    m2 = ei == e2

    t = jnp.exp(v2 - v1)
    w1 = 1.0 / (1.0 + t)
    w2 = t / (1.0 + t)

    oh = (m1 | m2).astype(jnp.float32)
    ri = lax.broadcasted_iota(jnp.int32, (TB, TB), 0)
    ci = lax.broadcasted_iota(jnp.int32, (TB, TB), 1)
    tril = (ri >= ci).astype(jnp.float32)
    csum = jnp.dot(tril, oh, preferred_element_type=jnp.float32)
    excl = csum - oh + carry_ref[...]
    carry_ref[...] = carry_ref[...] + jnp.sum(oh, axis=0, keepdims=True)

    s1 = jnp.sum(jnp.where(m1, excl, 0.0), axis=1, keepdims=True)
    s2 = jnp.sum(jnp.where(m2, excl, 0.0), axis=1, keepdims=True)
    ok1 = s1 < CAP
    ok2 = s2 < CAP
    d1_ref[...] = jnp.where(ok1, e1 * CAP + s1.astype(jnp.int32), -1)
    d2_ref[...] = jnp.where(ok2, e2 * CAP + s2.astype(jnp.int32), -1)
    w1_ref[...] = w1 * ok1.astype(jnp.float32)
    w2_ref[...] = w2 * ok2.astype(jnp.float32)


def _router(x_flat, noise_flat, Wg, bg, Wn, bn):
    return pl.pallas_call(
        _router_body,
        grid=(NTB,),
        in_specs=[
            pl.BlockSpec((TB, D), lambda i: (i, 0)),
            pl.BlockSpec((TB, E), lambda i: (i, 0)),
            pl.BlockSpec((E, D), lambda i: (0, 0)),
            pl.BlockSpec((1, E), lambda i: (0, 0)),
            pl.BlockSpec((E, D), lambda i: (0, 0)),
            pl.BlockSpec((1, E), lambda i: (0, 0)),
        ],
        out_specs=[
            pl.BlockSpec((TB, 1), lambda i: (i, 0)),
            pl.BlockSpec((TB, 1), lambda i: (i, 0)),
            pl.BlockSpec((TB, 1), lambda i: (i, 0)),
            pl.BlockSpec((TB, 1), lambda i: (i, 0)),
        ],
        out_shape=[
            jax.ShapeDtypeStruct((T, 1), jnp.int32),
            jax.ShapeDtypeStruct((T, 1), jnp.int32),
            jax.ShapeDtypeStruct((T, 1), jnp.float32),
            jax.ShapeDtypeStruct((T, 1), jnp.float32),
        ],
        scratch_shapes=[pltpu.VMEM((1, E), jnp.float32)],
    )(x_flat, noise_flat, Wg, bg.reshape(1, E), Wn, bn.reshape(1, E))


@functools.partial(
    pl.kernel,
    out_type=jax.ShapeDtypeStruct((P, D), jnp.float32),
    mesh=_SC_MESH,
    scratch_types=[
        pltpu.VMEM((ROWS_W,), jnp.int32),
        pltpu.VMEM((SCAN_CH,), jnp.int32),
        pltpu.VMEM((GCH, D), jnp.float32),
        pltpu.SemaphoreType.DMA,
    ],
    compiler_params=pltpu.CompilerParams(needs_layout_passes=False),
)
def _dispatch(x_hbm, dpair_hbm, xg_hbm, idx_v, dbuf, rows_v, sem):
    wid = lax.axis_index("s") * NSC_C + lax.axis_index("c")
    base = wid * ROWS_W
    lane = lax.iota(jnp.int32, 16)
    zero16 = jnp.zeros((16,), jnp.int32)
    for i in range(ROWS_W // 16):
        idx_v[pl.ds(i * 16, 16)] = zero16

    def scan_chunk(c, carry):
        pltpu.sync_copy(dpair_hbm.at[pl.ds(c * SCAN_CH, SCAN_CH)], dbuf)
        for j in range(SCAN_CH // 16):
            d = dbuf[pl.ds(j * 16, 16)]
            local = d - base
            msk = (local >= 0) & (local < ROWS_W)
            loc_c = jnp.clip(local, 0, ROWS_W - 1)
            p_vec = c * SCAN_CH + j * 16 + lane
            tok = p_vec & (T - 1)
            plsc.store_scatter(idx_v, [loc_c], tok, mask=msk)
        return carry

    lax.fori_loop(0, N_SCAN, scan_chunk, 0)

    for k in range(ROWS_W // GCH):
        pltpu.async_copy(
            x_hbm.at[idx_v.at[pl.ds(k * GCH, GCH)]], rows_v, sem).wait()
        pltpu.sync_copy(rows_v, xg_hbm.at[pl.ds(base + k * GCH, GCH)])


def _ffn_body(xg_ref, w1_ref, w3_ref, w2_ref, y_ref):
    h = pl.program_id(1)
    xg = xg_ref[...]
    a = jnp.dot(xg, w1_ref[0].T, preferred_element_type=jnp.float32)
    b = jnp.dot(xg, w3_ref[0].T, preferred_element_type=jnp.float32)
    hh = (a * (1.0 / (1.0 + jnp.exp(-a)))) * b
    y = jnp.dot(hh, w2_ref[0].T, preferred_element_type=jnp.float32)

    @pl.when(h == 0)
    def _():
        y_ref[...] = y

    @pl.when(h > 0)
    def _():
        y_ref[...] += y


def _ffn(xg, W1, W3, W2):
    return pl.pallas_call(
        _ffn_body,
        grid=(E, NHB),
        in_specs=[
            pl.BlockSpec((CAP, D), lambda e, h: (e, 0)),
            pl.BlockSpec((1, HB, D), lambda e, h: (e, h, 0)),
            pl.BlockSpec((1, HB, D), lambda e, h: (e, h, 0)),
            pl.BlockSpec((1, D, HB), lambda e, h: (e, 0, h)),
        ],
        out_specs=pl.BlockSpec((CAP, D), lambda e, h: (e, 0)),
        out_shape=jax.ShapeDtypeStruct((P, D), jnp.float32),
        compiler_params=pltpu.CompilerParams(
            dimension_semantics=("parallel", "arbitrary")),
    )(xg, W1, W3, W2)


@functools.partial(
    pl.kernel,
    out_type=jax.ShapeDtypeStruct((T, D), jnp.float32),
    mesh=_SC_MESH,
    scratch_types=[
        pltpu.VMEM((TOK_W,), jnp.int32),
        pltpu.VMEM((TOK_W,), jnp.int32),
        pltpu.VMEM((TOK_W,), jnp.float32),
        pltpu.VMEM((TOK_W,), jnp.float32),
        pltpu.VMEM((CCH,), jnp.int32),
        pltpu.VMEM((CCH,), jnp.int32),
        pltpu.VMEM((CCH, D), jnp.float32),
        pltpu.VMEM((CCH, D), jnp.float32),
        pltpu.VMEM((CCH, D), jnp.float32),
        pltpu.SemaphoreType.DMA,
        pltpu.SemaphoreType.DMA,
    ],
    compiler_params=pltpu.CompilerParams(needs_layout_passes=False),
)
def _combine(y_hbm, d1_hbm, d2_hbm, w1_hbm, w2_hbm, out_hbm,
             d1b, d2b, w1b, w2b, i1c, i2c, r1, r2, ob, sem1, sem2):
    wid = lax.axis_index("s") * NSC_C + lax.axis_index("c")
    base = wid * TOK_W
    pltpu.sync_copy(d1_hbm.at[pl.ds(base, TOK_W)], d1b)
    pltpu.sync_copy(d2_hbm.at[pl.ds(base, TOK_W)], d2b)
    pltpu.sync_copy(w1_hbm.at[pl.ds(base, TOK_W)], w1b)
    pltpu.sync_copy(w2_hbm.at[pl.ds(base, TOK_W)], w2b)
    for k in range(TOK_W // CCH):
        for j in range(CCH // 16):
            i1c[pl.ds(j * 16, 16)] = jnp.maximum(
                d1b[pl.ds(k * CCH + j * 16, 16)], 0)
            i2c[pl.ds(j * 16, 16)] = jnp.maximum(
                d2b[pl.ds(k * CCH + j * 16, 16)], 0)
        c1 = pltpu.async_copy(y_hbm.at[i1c], r1, sem1)
        c2 = pltpu.async_copy(y_hbm.at[i2c], r2, sem2)
        c1.wait()
        c2.wait()

        def row(i, carry):
            bidx = jnp.zeros((16,), jnp.int32) + (k * CCH + i)
            wv1 = plsc.load_gather(w1b, [bidx])
            wv2 = plsc.load_gather(w2b, [bidx])
            for v in range(D // 16):
                sl = pl.ds(v * 16, 16)
                ob[i, sl] = wv1 * r1[i, sl] + wv2 * r2[i, sl]
            return carry

        lax.fori_loop(0, CCH, row, 0)
        pltpu.sync_copy(ob, out_hbm.at[pl.ds(base + k * CCH, CCH)])


def kernel(x, noise, Wg, bg, Wn, bn, W1, W3, W2):
    b, s, d = x.shape
    xf = x.reshape(T, D)
    d1, d2, w1, w2 = _router(xf, noise.reshape(T, E), Wg, bg, Wn, bn)
    d1, d2, w1, w2 = d1[:, 0], d2[:, 0], w1[:, 0], w2[:, 0]
    dpair = jnp.concatenate([d1, d2])
    xg = _dispatch(xf, dpair)
    y = _ffn(xg, W1, W3, W2)
    out = _combine(y, d1, d2, w1, w2)
    return out.reshape(b, s, d)


# speedup vs baseline: 2.4541x; 1.1483x over previous
import functools
import jax, jax.numpy as jnp
from jax import lax
from jax.experimental import pallas as pl
from jax.experimental.pallas import tpu as pltpu
from jax.experimental.pallas import tpu_sc as plsc

B, S, D = 2, 2048, 1024
E, TOPK = 8, 2
HID = 2816
T = B * S
CAP = T * TOPK // E
P = T * TOPK
TB = 512
NTB = T // TB
HB = 1408
NHB = HID // HB
RB = 512
NRB = CAP // RB
XG_ROWS = P + RB

NSC_C, NSC_S, LANES = 2, 16, 16
NW = NSC_C * NSC_S
TOK_W = T // NW
DCH = 32
CCH = 16

_SC_MESH = plsc.VectorSubcoreMesh(core_axis_name="c", subcore_axis_name="s")


def _router_body(x_ref, noise_ref, wg_ref, bg_ref, wn_ref, bn_ref,
                 d_ref, w_ref, carry_ref):
    i = pl.program_id(0)

    @pl.when(i == 0)
    def _():
        carry_ref[...] = jnp.zeros_like(carry_ref)

    x = x_ref[...]
    lg = jnp.dot(x, wg_ref[...].T, preferred_element_type=jnp.float32) + bg_ref[...]
    ln = jnp.dot(x, wn_ref[...].T, preferred_element_type=jnp.float32) + bn_ref[...]
    sp = jnp.maximum(ln, 0.0) + jnp.log1p(jnp.exp(-jnp.abs(ln)))
    nz = lg + noise_ref[...] * sp

    ei = lax.broadcasted_iota(jnp.int32, (TB, E), 1)
    v1 = jnp.max(nz, axis=1, keepdims=True)
    e1 = jnp.min(jnp.where(nz == v1, ei, E), axis=1, keepdims=True)
    m1 = ei == e1
    nz2 = jnp.where(m1, -jnp.inf, nz)
    v2 = jnp.max(nz2, axis=1, keepdims=True)
    e2 = jnp.min(jnp.where(nz2 == v2, ei, E), axis=1, keepdims=True)
    m2 = ei == e2

    t = jnp.exp(v2 - v1)
    w1 = 1.0 / (1.0 + t)
    w2 = t / (1.0 + t)

    oh = (m1 | m2).astype(jnp.float32)
    ri = lax.broadcasted_iota(jnp.int32, (TB, TB), 0)
    ci = lax.broadcasted_iota(jnp.int32, (TB, TB), 1)
    tril = (ri >= ci).astype(jnp.float32)
    csum = jnp.dot(tril, oh, preferred_element_type=jnp.float32)
    excl = csum - oh + carry_ref[...]
    carry_ref[...] = carry_ref[...] + jnp.sum(oh, axis=0, keepdims=True)

    s1 = jnp.sum(jnp.where(m1, excl, 0.0), axis=1, keepdims=True)
    s2 = jnp.sum(jnp.where(m2, excl, 0.0), axis=1, keepdims=True)
    ok1 = s1 < CAP
    ok2 = s2 < CAP
    d1 = jnp.where(ok1, e1 * CAP + s1.astype(jnp.int32), -1)
    d2 = jnp.where(ok2, e2 * CAP + s2.astype(jnp.int32), -1)
    d_ref[0] = d1
    d_ref[1] = d2
    w_ref[0] = w1 * ok1.astype(jnp.float32)
    w_ref[1] = w2 * ok2.astype(jnp.float32)


def _router(x_flat, noise_flat, Wg, bg, Wn, bn):
    return pl.pallas_call(
        _router_body,
        grid=(NTB,),
        in_specs=[
            pl.BlockSpec((TB, D), lambda i: (i, 0)),
            pl.BlockSpec((TB, E), lambda i: (i, 0)),
            pl.BlockSpec((E, D), lambda i: (0, 0)),
            pl.BlockSpec((1, E), lambda i: (0, 0)),
            pl.BlockSpec((E, D), lambda i: (0, 0)),
            pl.BlockSpec((1, E), lambda i: (0, 0)),
        ],
        out_specs=[
            pl.BlockSpec((2, TB, 1), lambda i: (0, i, 0)),
            pl.BlockSpec((2, TB, 1), lambda i: (0, i, 0)),
        ],
        out_shape=[
            jax.ShapeDtypeStruct((2, T, 1), jnp.int32),
            jax.ShapeDtypeStruct((2, T, 1), jnp.float32),
        ],
        scratch_shapes=[pltpu.VMEM((1, E), jnp.float32)],
    )(x_flat, noise_flat, Wg, bg.reshape(1, E), Wn, bn.reshape(1, E))


@functools.partial(
    pl.kernel,
    out_type=jax.ShapeDtypeStruct((XG_ROWS, D), jnp.float32),
    mesh=_SC_MESH,
    scratch_types=[
        pltpu.VMEM((TOK_W,), jnp.int32),
        pltpu.VMEM((TOK_W,), jnp.int32),
        pltpu.VMEM((DCH,), jnp.int32),
        pltpu.VMEM((DCH,), jnp.int32),
        pltpu.VMEM((2, DCH, D), jnp.float32),
        pltpu.SemaphoreType.DMA,
        pltpu.SemaphoreType.DMA,
        pltpu.SemaphoreType.DMA,
    ],
    compiler_params=pltpu.CompilerParams(needs_layout_passes=False),
)
def _dispatch(x_hbm, dpair_hbm, xg_hbm, d1s, d2s, i1c, i2c, rows_v,
              ldsem, sem1, sem2):
    wid = lax.axis_index("s") * NSC_C + lax.axis_index("c")
    base = wid * TOK_W
    trash = P + wid
    pltpu.sync_copy(dpair_hbm.at[pl.ds(base, TOK_W)], d1s)
    pltpu.sync_copy(dpair_hbm.at[pl.ds(T + base, TOK_W)], d2s)

    nch = TOK_W // DCH
    lds = [pltpu.async_copy(
        x_hbm.at[pl.ds(base + c * DCH, DCH)], rows_v.at[c % 2], ldsem)
        for c in range(min(2, nch))]
    for c in range(nch):
        lds[c].wait()
        for j in range(DCH // 16):
            dv1 = d1s[pl.ds(c * DCH + j * 16, 16)]
            dv2 = d2s[pl.ds(c * DCH + j * 16, 16)]
            i1c[pl.ds(j * 16, 16)] = jnp.where(dv1 < 0, trash, dv1)
            i2c[pl.ds(j * 16, 16)] = jnp.where(dv2 < 0, trash, dv2)
        c1 = pltpu.async_copy(rows_v.at[c % 2], xg_hbm.at[i1c], sem1)
        c2 = pltpu.async_copy(rows_v.at[c % 2], xg_hbm.at[i2c], sem2)
        c1.wait()
        c2.wait()
        if c + 2 < nch:
            lds.append(pltpu.async_copy(
                x_hbm.at[pl.ds(base + (c + 2) * DCH, DCH)],
                rows_v.at[c % 2], ldsem))


def _ffn_body(xg_ref, w1_ref, w3_ref, w2_ref, y_ref):
    h = pl.program_id(2)
    xg = xg_ref[...].astype(jnp.bfloat16)
    w1 = w1_ref[0].astype(jnp.bfloat16)
    w3 = w3_ref[0].astype(jnp.bfloat16)
    w2 = w2_ref[0].astype(jnp.bfloat16)
    a = jnp.dot(xg, w1.T, preferred_element_type=jnp.float32)
    b = jnp.dot(xg, w3.T, preferred_element_type=jnp.float32)
    hh = ((a * (1.0 / (1.0 + jnp.exp(-a)))) * b).astype(jnp.bfloat16)
    y = jnp.dot(hh, w2.T, preferred_element_type=jnp.float32)

    @pl.when(h == 0)
    def _():
        y_ref[...] = y

    @pl.when(h > 0)
    def _():
        y_ref[...] += y


def _ffn(xg, W1, W3, W2):
    return pl.pallas_call(
        _ffn_body,
        grid=(E, NRB, NHB),
        in_specs=[
            pl.BlockSpec((RB, D), lambda e, r, h: (e * NRB + r, 0)),
            pl.BlockSpec((1, HB, D), lambda e, r, h: (e, h, 0)),
            pl.BlockSpec((1, HB, D), lambda e, r, h: (e, h, 0)),
            pl.BlockSpec((1, D, HB), lambda e, r, h: (e, 0, h)),
        ],
        out_specs=pl.BlockSpec((RB, D), lambda e, r, h: (e * NRB + r, 0)),
        out_shape=jax.ShapeDtypeStruct((P, D), jnp.float32),
        compiler_params=pltpu.CompilerParams(
            dimension_semantics=("parallel", "arbitrary", "arbitrary")),
    )(xg, W1, W3, W2)


@functools.partial(
    pl.kernel,
    out_type=jax.ShapeDtypeStruct((T, D), jnp.float32),
    mesh=_SC_MESH,
    scratch_types=[
        pltpu.VMEM((TOK_W,), jnp.int32),
        pltpu.VMEM((TOK_W,), jnp.int32),
        pltpu.VMEM((TOK_W,), jnp.float32),
        pltpu.VMEM((TOK_W,), jnp.float32),
        pltpu.VMEM((2, CCH), jnp.int32),
        pltpu.VMEM((2, CCH), jnp.int32),
        pltpu.VMEM((2, CCH, D), jnp.float32),
        pltpu.VMEM((2, CCH, D), jnp.float32),
        pltpu.VMEM((2, CCH, D), jnp.float32),
        pltpu.SemaphoreType.DMA,
        pltpu.SemaphoreType.DMA,
        pltpu.SemaphoreType.DMA,
    ],
    compiler_params=pltpu.CompilerParams(needs_layout_passes=False),
)
def _combine(y_hbm, dpair_hbm, wpair_hbm, out_hbm,
             d1b, d2b, w1b, w2b, i1c, i2c, r1, r2, ob, sem1, sem2, osem):
    wid = lax.axis_index("s") * NSC_C + lax.axis_index("c")
    base = wid * TOK_W
    pltpu.sync_copy(dpair_hbm.at[pl.ds(base, TOK_W)], d1b)
    pltpu.sync_copy(dpair_hbm.at[pl.ds(T + base, TOK_W)], d2b)
    pltpu.sync_copy(wpair_hbm.at[pl.ds(base, TOK_W)], w1b)
    pltpu.sync_copy(wpair_hbm.at[pl.ds(T + base, TOK_W)], w2b)

    nch = TOK_W // CCH

    def stage(k, buf):
        i1c[buf] = jnp.maximum(d1b[pl.ds(k * CCH, CCH)], 0)
        i2c[buf] = jnp.maximum(d2b[pl.ds(k * CCH, CCH)], 0)
        g1 = pltpu.async_copy(y_hbm.at[i1c.at[buf]], r1.at[buf], sem1)
        g2 = pltpu.async_copy(y_hbm.at[i2c.at[buf]], r2.at[buf], sem2)
        return g1, g2

    pend = [stage(0, 0)]
    writes = []
    for k in range(nch):
        buf = k % 2
        g1, g2 = pend[k]
        g1.wait()
        g2.wait()
        if k + 1 < nch:
            pend.append(stage(k + 1, (k + 1) % 2))
        if k >= 2:
            writes[k - 2].wait()

        def row(i, carry):
            bidx = jnp.zeros((16,), jnp.int32) + (k * CCH + i)
            wv1 = plsc.load_gather(w1b, [bidx])
            wv2 = plsc.load_gather(w2b, [bidx])
            for v in range(D // 16):
                sl = pl.ds(v * 16, 16)
                ob[buf, i, sl] = wv1 * r1[buf, i, sl] + wv2 * r2[buf, i, sl]
            return carry

        lax.fori_loop(0, CCH, row, 0)
        writes.append(pltpu.async_copy(
            ob.at[buf], out_hbm.at[pl.ds(base + k * CCH, CCH)], osem))
    writes[-2].wait()
    writes[-1].wait()


def kernel(x, noise, Wg, bg, Wn, bn, W1, W3, W2):
    b, s, d = x.shape
    xf = x.reshape(T, D)
    dpair, wpair = _router(xf, noise.reshape(T, E), Wg, bg, Wn, bn)
    dpair = dpair.reshape(2 * T)
    wpair = wpair.reshape(2 * T)
    xg = _dispatch(xf, dpair)
    y = _ffn(xg, W1, W3, W2)
    out = _combine(y, dpair, wpair)
    return out.reshape(b, s, d)
